# Initial kernel scaffold; baseline (speedup 1.0000x reference)
import jax
import jax.numpy as jnp
from jax.experimental import pallas as pl


def kernel(token_repr, t_frac, rare_mask, W1, b1, W2, b2):
    raise NotImplementedError("write your pallas kernel here")


# speedup vs baseline: 1.1891x; 1.1891x over previous
import numpy as np
import jax
import jax.numpy as jnp
from jax import lax
from jax.experimental import pallas as pl
from jax.experimental.pallas import tpu as pltpu

_D = 2048
_H = 512
_B = 4
_N = 4096
_TN = 512
_G = (_B * _N) // _TN
_K_RANK = 3277

_INTERPRET = False

_SQRT_HALF = np.float32(np.sqrt(0.5))

_ERFC_P = [2.326819970068386e-2, -1.387039388740657e-1, 3.687424674597105e-1,
           -5.824733027278666e-1, 6.210004764949774e-1, -4.944515323274145e-1,
           3.404879937665872e-1, -2.741127028184656e-1, 5.638259427386472e-1]
_ERFC_R = [-1.047766399936249e+1, 1.297719955372516e+1, -7.495518717768503e+0,
           2.921019019210786e+0, -1.015265279202700e+0, 4.218463358204948e-1,
           -2.820767439740514e-1, 5.641895067754075e-1]
_ERF_T = [7.853861353153693e-5, -8.010193625184903e-4, 5.188327685732524e-3,
          -2.685381193529856e-2, 1.128358514861418e-1, -3.761262582423300e-1,
          1.128379165726710e+0]
_MAXLOG = np.float32(88.72283905206835)


def _poly(y, coefs):
    p = jnp.zeros_like(y)
    for c in coefs:
        p = p * y + np.float32(c)
    return p


def _erfc_f32(x):
    abs_x = jnp.abs(x)
    w = -(x * x)
    z = jnp.exp(w)
    q = 1.0 / abs_x
    y2 = q * q
    p = jnp.where(abs_x < 2.0, _poly(y2, _ERFC_P), _poly(y2, _ERFC_R))
    yv = z * q * p
    y_clamp = jnp.where(w < -_MAXLOG, 0.0, yv)
    erfc_big = jnp.where(x < 0.0, 2.0 - y_clamp, y_clamp)
    erf_small = x * _poly(x * x, _ERF_T)
    return jnp.where(abs_x > 1.0, erfc_big, 1.0 - erf_small)


def _score_body(x_ref, w1t_ref, b1_ref, w2p_ref, b2_ref, o_ref):
    x = x_ref[0]
    h = jnp.dot(x.astype(jnp.bfloat16), w1t_ref[...].astype(jnp.bfloat16),
                preferred_element_type=jnp.float32)
    h = h + b1_ref[...]
    g = 0.5 * h * _erfc_f32(h * -_SQRT_HALF)
    logit = jnp.dot(g.astype(jnp.bfloat16), w2p_ref[...].astype(jnp.bfloat16),
                    preferred_element_type=jnp.float32)[:, 0]
    logit = logit + b2_ref[0, 0]
    o_ref[0, 0] = 1.0 / (1.0 + jnp.exp(-logit))


def _thr_body(sig_ref, t_ref, rare_ref, mask_ref, s_ref):
    t = t_ref[...][:, 0:1]
    s = sig_ref[...] * t
    s_ref[...] = s
    keys = lax.bitcast_convert_type(s, jnp.int32)

    def step(_, lohi):
        lo, hi = lohi
        mid = (lo + hi) >> 1
        cnt = jnp.sum((keys <= mid).astype(jnp.int32), axis=1, keepdims=True)
        ge = cnt >= _K_RANK
        return jnp.where(ge, lo, mid + 1), jnp.where(ge, mid, hi)

    lo0 = jnp.zeros((_B, 1), jnp.int32)
    hi0 = jnp.full((_B, 1), 0x3F800000, jnp.int32)
    _, hi = lax.fori_loop(0, 31, step, (lo0, hi0))
    thr = lax.bitcast_convert_type(hi, jnp.float32)
    keep = (s > thr) & (rare_ref[...] == 0)
    mask_ref[...] = keep.astype(jnp.int32)


def kernel(token_repr, t_frac, rare_mask, W1, b1, W2, b2):
    x = token_repr.reshape(_G, _TN, _D)
    w1t = W1.T
    b1r = b1.reshape(1, _H)
    w2p = jnp.zeros((_H, 128), W2.dtype).at[:, 0].set(W2[0])
    b2r = b2.reshape(1, 1)

    sig = pl.pallas_call(
        _score_body,
        grid=(_G,),
        in_specs=[
            pl.BlockSpec((1, _TN, _D), lambda i: (i, 0, 0)),
            pl.BlockSpec((_D, _H), lambda i: (0, 0)),
            pl.BlockSpec((1, _H), lambda i: (0, 0)),
            pl.BlockSpec((_H, 128), lambda i: (0, 0)),
            pl.BlockSpec(memory_space=pltpu.SMEM),
        ],
        out_specs=pl.BlockSpec((1, 1, _TN), lambda i: (i, 0, 0)),
        out_shape=jax.ShapeDtypeStruct((_G, 1, _TN), jnp.float32),
        interpret=_INTERPRET,
    )(x, w1t, b1r, w2p, b2r)

    sig = sig.reshape(_B, _N)
    t_b = jnp.broadcast_to(t_frac[:, None], (_B, 128))
    rare_i = rare_mask.astype(jnp.int32)

    mask_i, scores = pl.pallas_call(
        _thr_body,
        out_shape=(
            jax.ShapeDtypeStruct((_B, _N), jnp.int32),
            jax.ShapeDtypeStruct((_B, _N), jnp.float32),
        ),
        interpret=_INTERPRET,
    )(sig, t_b, rare_i)

    return mask_i.astype(jnp.bool_), scores


# speedup vs baseline: 1.2484x; 1.0499x over previous
import numpy as np
import jax
import jax.numpy as jnp
from jax import lax
from jax.experimental import pallas as pl
from jax.experimental.pallas import tpu as pltpu

_D = 2048
_H = 512
_B = 4
_N = 4096
_TN = 512
_G = (_B * _N) // _TN
_K_RANK = 3277

_INTERPRET = False

_SQRT_HALF = np.float32(np.sqrt(0.5))

_ERFC_P = [2.326819970068386e-2, -1.387039388740657e-1, 3.687424674597105e-1,
           -5.824733027278666e-1, 6.210004764949774e-1, -4.944515323274145e-1,
           3.404879937665872e-1, -2.741127028184656e-1, 5.638259427386472e-1]
_ERFC_R = [-1.047766399936249e+1, 1.297719955372516e+1, -7.495518717768503e+0,
           2.921019019210786e+0, -1.015265279202700e+0, 4.218463358204948e-1,
           -2.820767439740514e-1, 5.641895067754075e-1]
_ERF_T = [7.853861353153693e-5, -8.010193625184903e-4, 5.188327685732524e-3,
          -2.685381193529856e-2, 1.128358514861418e-1, -3.761262582423300e-1,
          1.128379165726710e+0]
_MAXLOG = np.float32(88.72283905206835)


def _poly(y, coefs):
    p = jnp.zeros_like(y)
    for c in coefs:
        p = p * y + np.float32(c)
    return p


def _erfc_f32(x):
    abs_x = jnp.abs(x)
    w = -(x * x)
    z = jnp.exp(w)
    q = 1.0 / abs_x
    y2 = q * q
    small = abs_x < 2.0
    p = jnp.zeros_like(x)
    for cp, cr in zip(_ERFC_P, [0.0] + _ERFC_R):
        p = p * y2 + jnp.where(small, np.float32(cp), np.float32(cr))
    yv = z * q * p
    y_clamp = jnp.where(w < -_MAXLOG, 0.0, yv)
    erfc_big = jnp.where(x < 0.0, 2.0 - y_clamp, y_clamp)
    erf_small = x * _poly(x * x, _ERF_T)
    return jnp.where(abs_x > 1.0, erfc_big, 1.0 - erf_small)


def _score_body(x_ref, w1t_ref, b1_ref, w2p_ref, b2_ref, o_ref):
    x = x_ref[0]
    h = jnp.dot(x.astype(jnp.bfloat16), w1t_ref[...].astype(jnp.bfloat16),
                preferred_element_type=jnp.float32)
    h = h + b1_ref[...]
    g = 0.5 * h * _erfc_f32(h * -_SQRT_HALF)
    logit = jnp.dot(g.astype(jnp.bfloat16), w2p_ref[...].astype(jnp.bfloat16),
                    preferred_element_type=jnp.float32)[:, 0]
    logit = logit + b2_ref[0, 0]
    o_ref[0, 0] = 1.0 / (1.0 + jnp.exp(-logit))


def _thr_body(sig_ref, t_ref, rare_ref, mask_ref, s_ref):
    t = t_ref[...][:, 0:1]
    s = sig_ref[...] * t
    s_ref[...] = s
    keys = lax.bitcast_convert_type(s, jnp.int32)

    def step(_, lohi):
        lo, hi = lohi
        mid = (lo + hi) >> 1
        cnt = jnp.sum((keys <= mid).astype(jnp.int32), axis=1, keepdims=True)
        ge = cnt >= _K_RANK
        return jnp.where(ge, lo, mid + 1), jnp.where(ge, mid, hi)

    lo0 = jnp.zeros((_B, 1), jnp.int32)
    hi0 = jnp.full((_B, 1), 0x3F800000, jnp.int32)
    _, hi = lax.fori_loop(0, 31, step, (lo0, hi0))
    thr = lax.bitcast_convert_type(hi, jnp.float32)
    keep = (s > thr) & (rare_ref[...] == 0)
    mask_ref[...] = keep.astype(jnp.int32)


def kernel(token_repr, t_frac, rare_mask, W1, b1, W2, b2):
    x = token_repr.reshape(_G, _TN, _D)
    w1t = W1.T
    b1r = b1.reshape(1, _H)
    w2p = jnp.zeros((_H, 128), W2.dtype).at[:, 0].set(W2[0])
    b2r = b2.reshape(1, 1)

    sig = pl.pallas_call(
        _score_body,
        grid=(_G,),
        in_specs=[
            pl.BlockSpec((1, _TN, _D), lambda i: (i, 0, 0)),
            pl.BlockSpec((_D, _H), lambda i: (0, 0)),
            pl.BlockSpec((1, _H), lambda i: (0, 0)),
            pl.BlockSpec((_H, 128), lambda i: (0, 0)),
            pl.BlockSpec(memory_space=pltpu.SMEM),
        ],
        out_specs=pl.BlockSpec((1, 1, _TN), lambda i: (i, 0, 0)),
        out_shape=jax.ShapeDtypeStruct((_G, 1, _TN), jnp.float32),
        interpret=_INTERPRET,
    )(x, w1t, b1r, w2p, b2r)

    sig = sig.reshape(_B, _N)
    t_b = jnp.broadcast_to(t_frac[:, None], (_B, 128))
    rare_i = rare_mask.astype(jnp.int32)

    mask_i, scores = pl.pallas_call(
        _thr_body,
        out_shape=(
            jax.ShapeDtypeStruct((_B, _N), jnp.int32),
            jax.ShapeDtypeStruct((_B, _N), jnp.float32),
        ),
        interpret=_INTERPRET,
    )(sig, t_b, rare_i)

    return mask_i.astype(jnp.bool_), scores


# speedup vs baseline: 1.2906x; 1.0338x over previous
import numpy as np
import jax
import jax.numpy as jnp
from jax import lax
from jax.experimental import pallas as pl
from jax.experimental.pallas import tpu as pltpu

_D = 2048
_H = 512
_B = 4
_N = 4096
_TN = 1024
_G = (_B * _N) // _TN
_K_RANK = 3277

_INTERPRET = False

_SQRT_HALF = np.float32(np.sqrt(0.5))

_ERFC_P = [2.326819970068386e-2, -1.387039388740657e-1, 3.687424674597105e-1,
           -5.824733027278666e-1, 6.210004764949774e-1, -4.944515323274145e-1,
           3.404879937665872e-1, -2.741127028184656e-1, 5.638259427386472e-1]
_ERFC_R = [-1.047766399936249e+1, 1.297719955372516e+1, -7.495518717768503e+0,
           2.921019019210786e+0, -1.015265279202700e+0, 4.218463358204948e-1,
           -2.820767439740514e-1, 5.641895067754075e-1]
_ERF_T = [7.853861353153693e-5, -8.010193625184903e-4, 5.188327685732524e-3,
          -2.685381193529856e-2, 1.128358514861418e-1, -3.761262582423300e-1,
          1.128379165726710e+0]
_MAXLOG = np.float32(88.72283905206835)


def _poly(y, coefs):
    p = jnp.zeros_like(y)
    for c in coefs:
        p = p * y + np.float32(c)
    return p


def _erfc_f32(x):
    abs_x = jnp.abs(x)
    w = -(x * x)
    z = jnp.exp(w)
    q = 1.0 / abs_x
    y2 = q * q
    small = abs_x < 2.0
    p = jnp.zeros_like(x)
    for cp, cr in zip(_ERFC_P, [0.0] + _ERFC_R):
        p = p * y2 + jnp.where(small, np.float32(cp), np.float32(cr))
    yv = z * q * p
    y_clamp = jnp.where(w < -_MAXLOG, 0.0, yv)
    erfc_big = jnp.where(x < 0.0, 2.0 - y_clamp, y_clamp)
    erf_small = x * _poly(x * x, _ERF_T)
    return jnp.where(abs_x > 1.0, erfc_big, 1.0 - erf_small)


def _score_body(x_ref, w1t_ref, b1_ref, w2p_ref, b2_ref, o_ref):
    x = x_ref[0]
    h = jnp.dot(x.astype(jnp.bfloat16), w1t_ref[...].astype(jnp.bfloat16),
                preferred_element_type=jnp.float32)
    h = h + b1_ref[...]
    g = 0.5 * h * _erfc_f32(h * -_SQRT_HALF)
    logit = jnp.dot(g.astype(jnp.bfloat16), w2p_ref[...].astype(jnp.bfloat16),
                    preferred_element_type=jnp.float32)[:, 0]
    logit = logit + b2_ref[0, 0]
    o_ref[0, 0] = 1.0 / (1.0 + jnp.exp(-logit))


def _thr_body(sig_ref, t_ref, rare_ref, mask_ref, s_ref):
    t = t_ref[...][:, 0:1]
    s = sig_ref[...] * t
    s_ref[...] = s
    keys = lax.bitcast_convert_type(s, jnp.int32)

    def step(_, lohi):
        lo, hi = lohi
        mid = (lo + hi) >> 1
        cnt = jnp.sum((keys <= mid).astype(jnp.int32), axis=1, keepdims=True)
        ge = cnt >= _K_RANK
        return jnp.where(ge, lo, mid + 1), jnp.where(ge, mid, hi)

    lo0 = jnp.zeros((_B, 1), jnp.int32)
    hi0 = jnp.full((_B, 1), 0x3F800000, jnp.int32)
    _, hi = lax.fori_loop(0, 31, step, (lo0, hi0))
    thr = lax.bitcast_convert_type(hi, jnp.float32)
    keep = (s > thr) & (rare_ref[...] == 0)
    mask_ref[...] = keep.astype(jnp.int32)


def kernel(token_repr, t_frac, rare_mask, W1, b1, W2, b2):
    x = token_repr.reshape(_G, _TN, _D)
    w1t = W1.T
    b1r = b1.reshape(1, _H)
    w2p = jnp.zeros((_H, 128), W2.dtype).at[:, 0].set(W2[0])
    b2r = b2.reshape(1, 1)

    sig = pl.pallas_call(
        _score_body,
        grid=(_G,),
        in_specs=[
            pl.BlockSpec((1, _TN, _D), lambda i: (i, 0, 0)),
            pl.BlockSpec((_D, _H), lambda i: (0, 0)),
            pl.BlockSpec((1, _H), lambda i: (0, 0)),
            pl.BlockSpec((_H, 128), lambda i: (0, 0)),
            pl.BlockSpec(memory_space=pltpu.SMEM),
        ],
        out_specs=pl.BlockSpec((1, 1, _TN), lambda i: (i, 0, 0)),
        out_shape=jax.ShapeDtypeStruct((_G, 1, _TN), jnp.float32),
        interpret=_INTERPRET,
    )(x, w1t, b1r, w2p, b2r)

    sig = sig.reshape(_B, _N)
    t_b = jnp.broadcast_to(t_frac[:, None], (_B, 128))
    rare_i = rare_mask.astype(jnp.int32)

    mask_i, scores = pl.pallas_call(
        _thr_body,
        out_shape=(
            jax.ShapeDtypeStruct((_B, _N), jnp.int32),
            jax.ShapeDtypeStruct((_B, _N), jnp.float32),
        ),
        interpret=_INTERPRET,
    )(sig, t_b, rare_i)

    return mask_i.astype(jnp.bool_), scores
